# baseline (speedup 1.0000x reference)
import functools

import jax
import jax.numpy as jnp
import numpy as np
from jax.experimental import pallas as pl
from jax.experimental.pallas import tpu as pltpu
from jax.experimental.pallas import tpu_sc as plsc

_ROT1 = (13, 15, 26, 6)
_ROT2 = (17, 29, 16, 24)
_K0 = 0
_K1 = 42
_K2 = _K0 ^ _K1 ^ 0x1BD11BDA
_TINY = float(jnp.finfo(jnp.float32).tiny)

_BLOCK = 8192
_CHUNK = 1024

_ROWS = 64
_VOCAB = 1000000
_TC_BLOCKS = 85
_C_TC = _TC_BLOCKS * _BLOCK
_SC_COLS = _VOCAB - _C_TC
_SC_CHUNK = 2048
_SC_FULL = _SC_COLS // _SC_CHUNK
_SC_TAIL = 640
_SC_PAD = _SC_FULL * _SC_CHUNK + _SC_TAIL
_NW = 32


def _rotl(x, r):
    return (x << jnp.uint32(r)) | (x >> jnp.uint32(32 - r))


def _threefry_xored(x1_init):
    ks = (jnp.uint32(_K0), jnp.uint32(_K1), jnp.uint32(_K2))
    x0 = jnp.full_like(x1_init, ks[0])
    x1 = x1_init
    rots = (_ROT1, _ROT2, _ROT1, _ROT2, _ROT1)
    for i, rset in enumerate(rots):
        for r in rset:
            x0 = x0 + x1
            x1 = _rotl(x1, r)
            x1 = x1 ^ x0
        j = i + 1
        x0 = x0 + ks[j % 3]
        x1 = x1 + ks[(j + 1) % 3] + jnp.uint32(j)
    return x0 ^ x1


def _uniform_from_bits(bits):
    float_bits = (bits >> jnp.uint32(9)) | jnp.uint32(0x3F800000)
    floats = jax.lax.bitcast_convert_type(float_bits, jnp.float32) - 1.0
    return jnp.maximum(jnp.float32(_TINY), floats + jnp.float32(_TINY))


def _chunk_scan(chunk_logits, base_chunk, off, col_iota):
    x1 = base_chunk + jnp.uint32(off)
    g = -jnp.log(-jnp.log(_uniform_from_bits(_threefry_xored(x1))))
    val = chunk_logits + g
    m = jnp.max(val, axis=1, keepdims=True)
    cand = jnp.where(val == m, col_iota, jnp.int32(0x7FFFFFFF))
    first = jnp.min(cand, axis=1, keepdims=True) + off
    return m, first


def _sample_block_kernel(base_ref, logits_ref, max_ref, idx_ref):
    j = pl.program_id(0)
    rows = logits_ref.shape[0]
    col_iota = jax.lax.broadcasted_iota(jnp.int32, (rows, _CHUNK), 1)

    @pl.when(j == 0)
    def _():
        max_ref[...] = jnp.full_like(max_ref, -jnp.inf)
        idx_ref[...] = jnp.zeros_like(idx_ref)

    m_run, i_run = None, None
    for k in range(_BLOCK // _CHUNK):
        off = j * _BLOCK + k * _CHUNK
        m, first = _chunk_scan(
            logits_ref[:, k * _CHUNK:(k + 1) * _CHUNK],
            base_ref[...], off, col_iota,
        )
        if m_run is None:
            m_run, i_run = m, first
        else:
            better = m > m_run
            m_run = jnp.where(better, m, m_run)
            i_run = jnp.where(better, first, i_run)
    better = m_run > max_ref[...]
    max_ref[...] = jnp.where(better, m_run, max_ref[...])
    idx_ref[...] = jnp.where(better, i_run, idx_ref[...])


def _sc_emit_chunk(buf, base_u32, ncols):
    iota16 = jax.lax.iota(jnp.uint32, 16)

    def vec_body(iv, _):
        off = iv * 64
        vb = base_u32 + jnp.uint32(off)
        for sub in range(4):
            x1 = jnp.full((16,), vb + jnp.uint32(sub * 16), jnp.uint32) + iota16
            u = _uniform_from_bits(_threefry_xored(x1))
            buf[pl.ds(off + sub * 16, 16)] = u
        return 0

    jax.lax.fori_loop(0, ncols // 64, vec_body, 0)


def _sc_uniform_kernel(out_hbm, buf0, buf1, sem0, sem1):
    c = jax.lax.axis_index("c")
    s = jax.lax.axis_index("s")
    wid = s * 2 + c
    r0 = wid * 2

    def copy(buf, sem, row, coff, ncols):
        return pltpu.make_async_copy(
            buf.at[pl.ds(0, ncols)],
            out_hbm.at[row, pl.ds(coff, ncols)],
            sem,
        )

    def chunk_pair(it, carry):
        half = _SC_FULL // 2
        row_inc = jnp.where(it >= half, 1, 0)
        row = r0 + row_inc
        ca = 2 * (it - row_inc * half) * _SC_CHUNK
        cb = ca + _SC_CHUNK
        base_a = (row * _VOCAB + _C_TC + ca + _K1).astype(jnp.uint32)

        @pl.when(it > 0)
        def _():
            copy(buf0, sem0, row, ca, _SC_CHUNK).wait()

        _sc_emit_chunk(buf0, base_a, _SC_CHUNK)
        copy(buf0, sem0, row, ca, _SC_CHUNK).start()

        @pl.when(it > 0)
        def _():
            copy(buf1, sem1, row, cb, _SC_CHUNK).wait()

        _sc_emit_chunk(buf1, base_a + jnp.uint32(_SC_CHUNK), _SC_CHUNK)
        copy(buf1, sem1, row, cb, _SC_CHUNK).start()
        return carry

    jax.lax.fori_loop(0, _SC_FULL, chunk_pair, 0)
    copy(buf0, sem0, r0, 0, _SC_CHUNK).wait()
    copy(buf1, sem1, r0, 0, _SC_CHUNK).wait()

    tail_off = _SC_FULL * _SC_CHUNK
    for t, (buf, sem) in enumerate(((buf0, sem0), (buf1, sem1))):
        row = r0 + t
        base = jnp.uint32((r0 + t) * _VOCAB + _C_TC + tail_off + _K1)
        _sc_emit_chunk(buf, base, _SC_TAIL)
        copy(buf, sem, row, tail_off, _SC_TAIL).start()
    copy(buf0, sem0, r0, tail_off, _SC_TAIL).wait()
    copy(buf1, sem1, r0 + 1, tail_off, _SC_TAIL).wait()


def _tail_block_kernel(u_ref, logits_ref, pmax_ref, pidx_ref, max_ref, idx_ref):
    j = pl.program_id(0)
    rows = u_ref.shape[0]
    col_iota = jax.lax.broadcasted_iota(jnp.int32, (rows, _CHUNK), 1)

    @pl.when(j == 0)
    def _():
        max_ref[...] = pmax_ref[...]
        idx_ref[...] = pidx_ref[...]

    m_run, i_run = None, None
    for k in range(_BLOCK // _CHUNK):
        off = j * _BLOCK + k * _CHUNK
        g = -jnp.log(-jnp.log(u_ref[:, k * _CHUNK:(k + 1) * _CHUNK]))
        val = logits_ref[:, k * _CHUNK:(k + 1) * _CHUNK] + g
        val = jnp.where(col_iota < _SC_COLS - off, val, -jnp.inf)
        m = jnp.max(val, axis=1, keepdims=True)
        cand = jnp.where(val == m, col_iota, jnp.int32(0x7FFFFFFF))
        first = jnp.min(cand, axis=1, keepdims=True) + (_C_TC + off)
        if m_run is None:
            m_run, i_run = m, first
        else:
            better = m > m_run
            m_run = jnp.where(better, m, m_run)
            i_run = jnp.where(better, first, i_run)
    better = m_run > max_ref[...]
    max_ref[...] = jnp.where(better, m_run, max_ref[...])
    idx_ref[...] = jnp.where(better, i_run, idx_ref[...])


@jax.jit
def kernel(logits):
    rows, vocab = logits.shape

    base = (
        np.arange(rows, dtype=np.uint32)[:, None] * np.uint32(vocab)
        + np.arange(_CHUNK, dtype=np.uint32)[None, :]
        + np.uint32(_K1)
    )

    sc_uniform = pl.kernel(
        _sc_uniform_kernel,
        out_type=jax.ShapeDtypeStruct((rows, _SC_PAD), jnp.float32),
        mesh=plsc.VectorSubcoreMesh(
            core_axis_name="c", subcore_axis_name="s",
            num_cores=2, num_subcores=16,
        ),
        scratch_types=[
            pltpu.VMEM((_SC_CHUNK,), jnp.float32),
            pltpu.VMEM((_SC_CHUNK,), jnp.float32),
            pltpu.SemaphoreType.DMA,
            pltpu.SemaphoreType.DMA,
        ],
        compiler_params=pltpu.CompilerParams(use_tc_tiling_on_sc=True),
    )
    u_tail = sc_uniform()

    max1, idx1 = pl.pallas_call(
        _sample_block_kernel,
        grid=(_TC_BLOCKS,),
        in_specs=[
            pl.BlockSpec((rows, _CHUNK), lambda j: (0, 0)),
            pl.BlockSpec((rows, _BLOCK), lambda j: (0, j)),
        ],
        out_specs=[
            pl.BlockSpec((rows, 1), lambda j: (0, 0)),
            pl.BlockSpec((rows, 1), lambda j: (0, 0)),
        ],
        out_shape=[
            jax.ShapeDtypeStruct((rows, 1), jnp.float32),
            jax.ShapeDtypeStruct((rows, 1), jnp.int32),
        ],
        compiler_params=pltpu.CompilerParams(
            dimension_semantics=("arbitrary",),
        ),
    )(jnp.asarray(base), logits)

    tail_blocks = (_SC_PAD + _BLOCK - 1) // _BLOCK
    _, idx = pl.pallas_call(
        _tail_block_kernel,
        grid=(tail_blocks,),
        in_specs=[
            pl.BlockSpec((rows, _BLOCK), lambda j: (0, j)),
            pl.BlockSpec((rows, _BLOCK), lambda j: (0, j + _TC_BLOCKS)),
            pl.BlockSpec((rows, 1), lambda j: (0, 0)),
            pl.BlockSpec((rows, 1), lambda j: (0, 0)),
        ],
        out_specs=[
            pl.BlockSpec((rows, 1), lambda j: (0, 0)),
            pl.BlockSpec((rows, 1), lambda j: (0, 0)),
        ],
        out_shape=[
            jax.ShapeDtypeStruct((rows, 1), jnp.float32),
            jax.ShapeDtypeStruct((rows, 1), jnp.int32),
        ],
        compiler_params=pltpu.CompilerParams(
            dimension_semantics=("arbitrary",),
        ),
    )(u_tail, logits, max1, idx1)
    return idx.reshape(rows)


# speedup vs baseline: 1.0704x; 1.0704x over previous
import functools

import jax
import jax.numpy as jnp
import numpy as np
from jax.experimental import pallas as pl
from jax.experimental.pallas import tpu as pltpu
from jax.experimental.pallas import tpu_sc as plsc

_ROT1 = (13, 15, 26, 6)
_ROT2 = (17, 29, 16, 24)
_K0 = 0
_K1 = 42
_K2 = _K0 ^ _K1 ^ 0x1BD11BDA
_TINY = float(jnp.finfo(jnp.float32).tiny)

_BLOCK = 8192
_CHUNK = 1024

_ROWS = 64
_VOCAB = 1000000
_TC_BLOCKS = 88
_C_TC = _TC_BLOCKS * _BLOCK
_SC_COLS = _VOCAB - _C_TC
_SC_CHUNK = 2048
_SC_FULL = _SC_COLS // _SC_CHUNK
_SC_TAIL = 640
_SC_PAD = _SC_FULL * _SC_CHUNK + _SC_TAIL
_NW = 32


def _rotl(x, r):
    return (x << jnp.uint32(r)) | (x >> jnp.uint32(32 - r))


def _threefry_xored(x1_init):
    ks = (jnp.uint32(_K0), jnp.uint32(_K1), jnp.uint32(_K2))
    x0 = jnp.full_like(x1_init, ks[0])
    x1 = x1_init
    rots = (_ROT1, _ROT2, _ROT1, _ROT2, _ROT1)
    for i, rset in enumerate(rots):
        for r in rset:
            x0 = x0 + x1
            x1 = _rotl(x1, r)
            x1 = x1 ^ x0
        j = i + 1
        x0 = x0 + ks[j % 3]
        x1 = x1 + ks[(j + 1) % 3] + jnp.uint32(j)
    return x0 ^ x1


def _uniform_from_bits(bits):
    float_bits = (bits >> jnp.uint32(9)) | jnp.uint32(0x3F800000)
    floats = jax.lax.bitcast_convert_type(float_bits, jnp.float32) - 1.0
    return jnp.maximum(jnp.float32(_TINY), floats + jnp.float32(_TINY))


def _chunk_scan(chunk_logits, base_chunk, off, col_iota):
    x1 = base_chunk + jnp.uint32(off)
    g = -jnp.log(-jnp.log(_uniform_from_bits(_threefry_xored(x1))))
    val = chunk_logits + g
    m = jnp.max(val, axis=1, keepdims=True)
    cand = jnp.where(val == m, col_iota, jnp.int32(0x7FFFFFFF))
    first = jnp.min(cand, axis=1, keepdims=True) + off
    return m, first


def _sample_block_kernel(base_ref, logits_ref, max_ref, idx_ref):
    j = pl.program_id(0)
    rows = logits_ref.shape[0]
    col_iota = jax.lax.broadcasted_iota(jnp.int32, (rows, _CHUNK), 1)

    @pl.when(j == 0)
    def _():
        max_ref[...] = jnp.full_like(max_ref, -jnp.inf)
        idx_ref[...] = jnp.zeros_like(idx_ref)

    m_run, i_run = None, None
    for k in range(_BLOCK // _CHUNK):
        off = j * _BLOCK + k * _CHUNK
        m, first = _chunk_scan(
            logits_ref[:, k * _CHUNK:(k + 1) * _CHUNK],
            base_ref[...], off, col_iota,
        )
        if m_run is None:
            m_run, i_run = m, first
        else:
            better = m > m_run
            m_run = jnp.where(better, m, m_run)
            i_run = jnp.where(better, first, i_run)
    better = m_run > max_ref[...]
    max_ref[...] = jnp.where(better, m_run, max_ref[...])
    idx_ref[...] = jnp.where(better, i_run, idx_ref[...])


def _sc_emit_chunk(buf, base_u32, ncols):
    iota16 = jax.lax.iota(jnp.uint32, 16)

    def vec_body(iv, _):
        off = iv * 64
        vb = base_u32 + jnp.uint32(off)
        for sub in range(4):
            x1 = jnp.full((16,), vb + jnp.uint32(sub * 16), jnp.uint32) + iota16
            u = _uniform_from_bits(_threefry_xored(x1))
            buf[pl.ds(off + sub * 16, 16)] = u
        return 0

    jax.lax.fori_loop(0, ncols // 64, vec_body, 0)


def _sc_uniform_kernel(out_hbm, buf0, buf1, sem0, sem1):
    c = jax.lax.axis_index("c")
    s = jax.lax.axis_index("s")
    wid = s * 2 + c
    r0 = wid * 2

    def copy(buf, sem, row, coff, ncols):
        return pltpu.make_async_copy(
            buf.at[pl.ds(0, ncols)],
            out_hbm.at[row, pl.ds(coff, ncols)],
            sem,
        )

    def chunk_pair(it, carry):
        half = _SC_FULL // 2
        row_inc = jnp.where(it >= half, 1, 0)
        row = r0 + row_inc
        ca = 2 * (it - row_inc * half) * _SC_CHUNK
        cb = ca + _SC_CHUNK
        base_a = (row * _VOCAB + _C_TC + ca + _K1).astype(jnp.uint32)

        @pl.when(it > 0)
        def _():
            copy(buf0, sem0, row, ca, _SC_CHUNK).wait()

        _sc_emit_chunk(buf0, base_a, _SC_CHUNK)
        copy(buf0, sem0, row, ca, _SC_CHUNK).start()

        @pl.when(it > 0)
        def _():
            copy(buf1, sem1, row, cb, _SC_CHUNK).wait()

        _sc_emit_chunk(buf1, base_a + jnp.uint32(_SC_CHUNK), _SC_CHUNK)
        copy(buf1, sem1, row, cb, _SC_CHUNK).start()
        return carry

    jax.lax.fori_loop(0, _SC_FULL, chunk_pair, 0)
    copy(buf0, sem0, r0, 0, _SC_CHUNK).wait()
    copy(buf1, sem1, r0, 0, _SC_CHUNK).wait()

    tail_off = _SC_FULL * _SC_CHUNK
    for t, (buf, sem) in enumerate(((buf0, sem0), (buf1, sem1))):
        row = r0 + t
        base = jnp.uint32((r0 + t) * _VOCAB + _C_TC + tail_off + _K1)
        _sc_emit_chunk(buf, base, _SC_TAIL)
        copy(buf, sem, row, tail_off, _SC_TAIL).start()
    copy(buf0, sem0, r0, tail_off, _SC_TAIL).wait()
    copy(buf1, sem1, r0 + 1, tail_off, _SC_TAIL).wait()


def _tail_block_kernel(u_ref, logits_ref, pmax_ref, pidx_ref, max_ref, idx_ref):
    j = pl.program_id(0)
    rows = u_ref.shape[0]
    col_iota = jax.lax.broadcasted_iota(jnp.int32, (rows, _CHUNK), 1)

    @pl.when(j == 0)
    def _():
        max_ref[...] = pmax_ref[...]
        idx_ref[...] = pidx_ref[...]

    m_run, i_run = None, None
    for k in range(_BLOCK // _CHUNK):
        off = j * _BLOCK + k * _CHUNK
        g = -jnp.log(-jnp.log(u_ref[:, k * _CHUNK:(k + 1) * _CHUNK]))
        val = logits_ref[:, k * _CHUNK:(k + 1) * _CHUNK] + g
        val = jnp.where(col_iota < _SC_COLS - off, val, -jnp.inf)
        m = jnp.max(val, axis=1, keepdims=True)
        cand = jnp.where(val == m, col_iota, jnp.int32(0x7FFFFFFF))
        first = jnp.min(cand, axis=1, keepdims=True) + (_C_TC + off)
        if m_run is None:
            m_run, i_run = m, first
        else:
            better = m > m_run
            m_run = jnp.where(better, m, m_run)
            i_run = jnp.where(better, first, i_run)
    better = m_run > max_ref[...]
    max_ref[...] = jnp.where(better, m_run, max_ref[...])
    idx_ref[...] = jnp.where(better, i_run, idx_ref[...])


@jax.jit
def kernel(logits):
    rows, vocab = logits.shape

    base = (
        np.arange(rows, dtype=np.uint32)[:, None] * np.uint32(vocab)
        + np.arange(_CHUNK, dtype=np.uint32)[None, :]
        + np.uint32(_K1)
    )

    sc_uniform = pl.kernel(
        _sc_uniform_kernel,
        out_type=jax.ShapeDtypeStruct((rows, _SC_PAD), jnp.float32),
        mesh=plsc.VectorSubcoreMesh(
            core_axis_name="c", subcore_axis_name="s",
            num_cores=2, num_subcores=16,
        ),
        scratch_types=[
            pltpu.VMEM((_SC_CHUNK,), jnp.float32),
            pltpu.VMEM((_SC_CHUNK,), jnp.float32),
            pltpu.SemaphoreType.DMA,
            pltpu.SemaphoreType.DMA,
        ],
        compiler_params=pltpu.CompilerParams(use_tc_tiling_on_sc=True),
    )
    u_tail = sc_uniform()

    max1, idx1 = pl.pallas_call(
        _sample_block_kernel,
        grid=(_TC_BLOCKS,),
        in_specs=[
            pl.BlockSpec((rows, _CHUNK), lambda j: (0, 0)),
            pl.BlockSpec((rows, _BLOCK), lambda j: (0, j)),
        ],
        out_specs=[
            pl.BlockSpec((rows, 1), lambda j: (0, 0)),
            pl.BlockSpec((rows, 1), lambda j: (0, 0)),
        ],
        out_shape=[
            jax.ShapeDtypeStruct((rows, 1), jnp.float32),
            jax.ShapeDtypeStruct((rows, 1), jnp.int32),
        ],
        compiler_params=pltpu.CompilerParams(
            dimension_semantics=("arbitrary",),
        ),
    )(jnp.asarray(base), logits)

    tail_blocks = (_SC_PAD + _BLOCK - 1) // _BLOCK
    _, idx = pl.pallas_call(
        _tail_block_kernel,
        grid=(tail_blocks,),
        in_specs=[
            pl.BlockSpec((rows, _BLOCK), lambda j: (0, j)),
            pl.BlockSpec((rows, _BLOCK), lambda j: (0, j + _TC_BLOCKS)),
            pl.BlockSpec((rows, 1), lambda j: (0, 0)),
            pl.BlockSpec((rows, 1), lambda j: (0, 0)),
        ],
        out_specs=[
            pl.BlockSpec((rows, 1), lambda j: (0, 0)),
            pl.BlockSpec((rows, 1), lambda j: (0, 0)),
        ],
        out_shape=[
            jax.ShapeDtypeStruct((rows, 1), jnp.float32),
            jax.ShapeDtypeStruct((rows, 1), jnp.int32),
        ],
        compiler_params=pltpu.CompilerParams(
            dimension_semantics=("arbitrary",),
        ),
    )(u_tail, logits, max1, idx1)
    return idx.reshape(rows)


# speedup vs baseline: 1.0705x; 1.0000x over previous
import jax
import jax.numpy as jnp
import numpy as np
from jax.experimental import pallas as pl
from jax.experimental.pallas import tpu as pltpu
from jax.experimental.pallas import tpu_sc as plsc

_ROT1 = (13, 15, 26, 6)
_ROT2 = (17, 29, 16, 24)
_K0 = 0
_K1 = 42
_K2 = _K0 ^ _K1 ^ 0x1BD11BDA
_TINY = float(jnp.finfo(jnp.float32).tiny)

_BLOCK = 8192
_CHUNK = 1024

_VOCAB = 1000000
_TC_BLOCKS = 88
_C_TC = _TC_BLOCKS * _BLOCK
_SC_COLS = _VOCAB - _C_TC
_SC_CHUNK = 2048
_SC_FULL = _SC_COLS // _SC_CHUNK
_SC_TAIL = 640
_SC_PAD = _SC_FULL * _SC_CHUNK + _SC_TAIL


def _rotl(x, r):
    return (x << jnp.uint32(r)) | (x >> jnp.uint32(32 - r))


def _threefry_xored(x1_init):
    ks = (jnp.uint32(_K0), jnp.uint32(_K1), jnp.uint32(_K2))
    x0 = jnp.full_like(x1_init, ks[0])
    x1 = x1_init
    rots = (_ROT1, _ROT2, _ROT1, _ROT2, _ROT1)
    for i, rset in enumerate(rots):
        for r in rset:
            x0 = x0 + x1
            x1 = _rotl(x1, r)
            x1 = x1 ^ x0
        j = i + 1
        x0 = x0 + ks[j % 3]
        x1 = x1 + ks[(j + 1) % 3] + jnp.uint32(j)
    return x0 ^ x1


def _uniform_from_bits(bits):
    float_bits = (bits >> jnp.uint32(9)) | jnp.uint32(0x3F800000)
    floats = jax.lax.bitcast_convert_type(float_bits, jnp.float32) - 1.0
    return jnp.maximum(jnp.float32(_TINY), floats + jnp.float32(_TINY))


def _chunk_scan(chunk_logits, base_chunk, off, col_iota):
    x1 = base_chunk + jnp.uint32(off)
    g = -jnp.log(-jnp.log(_uniform_from_bits(_threefry_xored(x1))))
    val = chunk_logits + g
    m = jnp.max(val, axis=1, keepdims=True)
    cand = jnp.where(val == m, col_iota, jnp.int32(0x7FFFFFFF))
    first = jnp.min(cand, axis=1, keepdims=True) + off
    return m, first


def _sample_block_kernel(base_ref, logits_ref, max_ref, idx_ref):
    j = pl.program_id(0)
    rows = logits_ref.shape[0]
    col_iota = jax.lax.broadcasted_iota(jnp.int32, (rows, _CHUNK), 1)

    @pl.when(j == 0)
    def _():
        max_ref[...] = jnp.full_like(max_ref, -jnp.inf)
        idx_ref[...] = jnp.zeros_like(idx_ref)

    m_run, i_run = None, None
    for k in range(_BLOCK // _CHUNK):
        off = j * _BLOCK + k * _CHUNK
        m, first = _chunk_scan(
            logits_ref[:, k * _CHUNK:(k + 1) * _CHUNK],
            base_ref[...], off, col_iota,
        )
        if m_run is None:
            m_run, i_run = m, first
        else:
            better = m > m_run
            m_run = jnp.where(better, m, m_run)
            i_run = jnp.where(better, first, i_run)
    better = m_run > max_ref[...]
    max_ref[...] = jnp.where(better, m_run, max_ref[...])
    idx_ref[...] = jnp.where(better, i_run, idx_ref[...])


def _sc_emit_chunk(buf, base_u32, ncols):
    iota16 = jax.lax.iota(jnp.uint32, 16)

    def vec_body(iv, _):
        off = iv * 64
        vb = base_u32 + jnp.uint32(off)
        for sub in range(4):
            x1 = jnp.full((16,), vb + jnp.uint32(sub * 16), jnp.uint32) + iota16
            u = _uniform_from_bits(_threefry_xored(x1))
            buf[pl.ds(off + sub * 16, 16)] = u
        return 0

    jax.lax.fori_loop(0, ncols // 64, vec_body, 0)


def _sc_uniform_kernel(out_hbm, buf0, buf1, sem0, sem1):
    c = jax.lax.axis_index("c")
    s = jax.lax.axis_index("s")
    wid = s * 2 + c
    r0 = wid * 2

    def copy(buf, sem, row, coff, ncols):
        return pltpu.make_async_copy(
            buf.at[pl.ds(0, ncols)],
            out_hbm.at[row, pl.ds(coff, ncols)],
            sem,
        )

    def chunk_pair(it, carry):
        half = _SC_FULL // 2
        row_inc = jnp.where(it >= half, 1, 0)
        row = r0 + row_inc
        ca = 2 * (it - row_inc * half) * _SC_CHUNK
        cb = ca + _SC_CHUNK
        base_a = (row * _VOCAB + _C_TC + ca + _K1).astype(jnp.uint32)

        @pl.when(it > 0)
        def _():
            copy(buf0, sem0, row, ca, _SC_CHUNK).wait()

        _sc_emit_chunk(buf0, base_a, _SC_CHUNK)
        copy(buf0, sem0, row, ca, _SC_CHUNK).start()

        @pl.when(it > 0)
        def _():
            copy(buf1, sem1, row, cb, _SC_CHUNK).wait()

        _sc_emit_chunk(buf1, base_a + jnp.uint32(_SC_CHUNK), _SC_CHUNK)
        copy(buf1, sem1, row, cb, _SC_CHUNK).start()
        return carry

    jax.lax.fori_loop(0, _SC_FULL, chunk_pair, 0)
    copy(buf0, sem0, r0, 0, _SC_CHUNK).wait()
    copy(buf1, sem1, r0, 0, _SC_CHUNK).wait()

    tail_off = _SC_FULL * _SC_CHUNK
    for t, (buf, sem) in enumerate(((buf0, sem0), (buf1, sem1))):
        row = r0 + t
        base = jnp.uint32((r0 + t) * _VOCAB + _C_TC + tail_off + _K1)
        _sc_emit_chunk(buf, base, _SC_TAIL)
        copy(buf, sem, row, tail_off, _SC_TAIL).start()
    copy(buf0, sem0, r0, tail_off, _SC_TAIL).wait()
    copy(buf1, sem1, r0 + 1, tail_off, _SC_TAIL).wait()


def _tail_block_kernel(u_ref, logits_ref, pmax_ref, pidx_ref, max_ref, idx_ref):
    j = pl.program_id(0)
    rows = u_ref.shape[0]
    col_iota = jax.lax.broadcasted_iota(jnp.int32, (rows, _CHUNK), 1)

    @pl.when(j == 0)
    def _():
        max_ref[...] = pmax_ref[...]
        idx_ref[...] = pidx_ref[...]

    m_run, i_run = None, None
    for k in range(_BLOCK // _CHUNK):
        off = j * _BLOCK + k * _CHUNK
        g = -jnp.log(-jnp.log(u_ref[:, k * _CHUNK:(k + 1) * _CHUNK]))
        val = logits_ref[:, k * _CHUNK:(k + 1) * _CHUNK] + g
        val = jnp.where(col_iota < _SC_COLS - off, val, -jnp.inf)
        m = jnp.max(val, axis=1, keepdims=True)
        cand = jnp.where(val == m, col_iota, jnp.int32(0x7FFFFFFF))
        first = jnp.min(cand, axis=1, keepdims=True) + (_C_TC + off)
        if m_run is None:
            m_run, i_run = m, first
        else:
            better = m > m_run
            m_run = jnp.where(better, m, m_run)
            i_run = jnp.where(better, first, i_run)
    better = m_run > max_ref[...]
    max_ref[...] = jnp.where(better, m_run, max_ref[...])
    idx_ref[...] = jnp.where(better, i_run, idx_ref[...])


@jax.jit
def kernel(logits):
    rows, vocab = logits.shape

    base = (
        np.arange(rows, dtype=np.uint32)[:, None] * np.uint32(vocab)
        + np.arange(_CHUNK, dtype=np.uint32)[None, :]
        + np.uint32(_K1)
    )

    sc_uniform = pl.kernel(
        _sc_uniform_kernel,
        out_type=jax.ShapeDtypeStruct((rows, _SC_PAD), jnp.float32),
        mesh=plsc.VectorSubcoreMesh(
            core_axis_name="c", subcore_axis_name="s",
            num_cores=2, num_subcores=16,
        ),
        scratch_types=[
            pltpu.VMEM((_SC_CHUNK,), jnp.float32),
            pltpu.VMEM((_SC_CHUNK,), jnp.float32),
            pltpu.SemaphoreType.DMA,
            pltpu.SemaphoreType.DMA,
        ],
        compiler_params=pltpu.CompilerParams(use_tc_tiling_on_sc=True),
    )
    u_tail = sc_uniform()

    max1, idx1 = pl.pallas_call(
        _sample_block_kernel,
        grid=(_TC_BLOCKS,),
        in_specs=[
            pl.BlockSpec((rows, _CHUNK), lambda j: (0, 0)),
            pl.BlockSpec((rows, _BLOCK), lambda j: (0, j)),
        ],
        out_specs=[
            pl.BlockSpec((rows, 1), lambda j: (0, 0)),
            pl.BlockSpec((rows, 1), lambda j: (0, 0)),
        ],
        out_shape=[
            jax.ShapeDtypeStruct((rows, 1), jnp.float32),
            jax.ShapeDtypeStruct((rows, 1), jnp.int32),
        ],
        compiler_params=pltpu.CompilerParams(
            dimension_semantics=("arbitrary",),
        ),
    )(jnp.asarray(base), logits)

    tail_blocks = (_SC_PAD + _BLOCK - 1) // _BLOCK
    _, idx = pl.pallas_call(
        _tail_block_kernel,
        grid=(tail_blocks,),
        in_specs=[
            pl.BlockSpec((rows, _BLOCK), lambda j: (0, j)),
            pl.BlockSpec((rows, _BLOCK), lambda j: (0, j + _TC_BLOCKS)),
            pl.BlockSpec((rows, 1), lambda j: (0, 0)),
            pl.BlockSpec((rows, 1), lambda j: (0, 0)),
        ],
        out_specs=[
            pl.BlockSpec((rows, 1), lambda j: (0, 0)),
            pl.BlockSpec((rows, 1), lambda j: (0, 0)),
        ],
        out_shape=[
            jax.ShapeDtypeStruct((rows, 1), jnp.float32),
            jax.ShapeDtypeStruct((rows, 1), jnp.int32),
        ],
        compiler_params=pltpu.CompilerParams(
            dimension_semantics=("arbitrary",),
        ),
    )(u_tail, logits, max1, idx1)
    return idx.reshape(rows)
